# Initial kernel scaffold; baseline (speedup 1.0000x reference)
import jax
import jax.numpy as jnp
from jax.experimental import pallas as pl


def kernel(inputs, edge_index, W1, b1, W2, b2):
    raise NotImplementedError("write your pallas kernel here")


# speedup vs baseline: 11.3503x; 11.3503x over previous
import functools

import jax
import jax.numpy as jnp
from jax import lax
from jax.experimental import pallas as pl
from jax.experimental.pallas import tpu as pltpu
from jax.experimental.pallas import tpu_sc as plsc

N = 10000
D = 128
NC = 2
NS = 16
NW = NC * NS
CH = 128
N_PAD = 10240
ROWS_PER_SUB = N_PAD // NS


def _mesh():
    return plsc.VectorSubcoreMesh(core_axis_name="c", subcore_axis_name="s")


def _zero_vmem(ref, rows, width):
    per_row = width // 16

    def body(i, carry):
        ref[i // per_row, pl.ds((i % per_row) * 16, 16)] = jnp.zeros((16,), jnp.float32)
        return carry

    lax.fori_loop(0, rows * per_row, body, 0)


def _make_deg_kernel(n_chunks):
    @functools.partial(
        pl.kernel,
        out_type=jax.ShapeDtypeStruct((NC, N_PAD, D), jnp.float32),
        mesh=_mesh(),
        scratch_types=[
            pltpu.VMEM((n_chunks, CH), jnp.int32),
            pltpu.VMEM((CH, D), jnp.float32),
            pltpu.VMEM_SHARED((N_PAD, D), jnp.float32),
        ],
    )
    def deg_kernel(dst_hbm, out, dst_v, ones_v, acc):
        c = lax.axis_index("c")
        s = lax.axis_index("s")
        wid = s * NC + c
        _zero_vmem(ones_v, CH, D)
        for k in range(ROWS_PER_SUB // CH):
            pltpu.sync_copy(ones_v, acc.at[pl.ds(s * ROWS_PER_SUB + k * CH, CH)])

        def fill_ones(i, carry):
            ones_v[i // 8, pl.ds((i % 8) * 16, 16)] = jnp.ones((16,), jnp.float32)
            return carry

        lax.fori_loop(0, CH * 8, fill_ones, 0)
        pltpu.sync_copy(dst_hbm.at[wid], dst_v)
        plsc.subcore_barrier()

        def body(j, carry):
            pltpu.sync_copy(ones_v, acc.at[dst_v.at[j]], add=True)
            return carry

        lax.fori_loop(0, n_chunks, body, 0)
        plsc.subcore_barrier()
        lo = s * ROWS_PER_SUB
        pltpu.sync_copy(acc.at[pl.ds(lo, ROWS_PER_SUB)],
                        out.at[c, pl.ds(lo, ROWS_PER_SUB)])

    return deg_kernel


def _make_edge_kernel(n_chunks):
    @functools.partial(
        pl.kernel,
        out_type=jax.ShapeDtypeStruct((NC, N_PAD, D), jnp.float32),
        mesh=_mesh(),
        scratch_types=[
            pltpu.VMEM((n_chunks, CH), jnp.int32),
            pltpu.VMEM((n_chunks, CH), jnp.int32),
            pltpu.VMEM((CH, D), jnp.float32),
            pltpu.VMEM_SHARED((N_PAD, D), jnp.float32),
            pltpu.SemaphoreType.DMA,
        ],
    )
    def edge_kernel(hs_hbm, src_hbm, dst_hbm, out, src_v, dst_v, rows_v, acc_sh, sem):
        c = lax.axis_index("c")
        s = lax.axis_index("s")
        wid = s * NC + c
        _zero_vmem(rows_v, CH, D)
        for k in range(ROWS_PER_SUB // CH):
            pltpu.sync_copy(rows_v, acc_sh.at[pl.ds(s * ROWS_PER_SUB + k * CH, CH)])
        pltpu.sync_copy(src_hbm.at[wid], src_v)
        pltpu.sync_copy(dst_hbm.at[wid], dst_v)
        plsc.subcore_barrier()

        def body(j, carry):
            pltpu.async_copy(hs_hbm.at[src_v.at[j]], rows_v, sem).wait()
            pltpu.sync_copy(rows_v, acc_sh.at[dst_v.at[j]], add=True)
            return carry

        lax.fori_loop(0, n_chunks, body, 0)
        plsc.subcore_barrier()
        lo = s * ROWS_PER_SUB
        pltpu.sync_copy(acc_sh.at[pl.ds(lo, ROWS_PER_SUB)],
                        out.at[c, pl.ds(lo, ROWS_PER_SUB)])

    return edge_kernel


BS = 1000


def _row_spec(width):
    return pl.BlockSpec((BS, width), lambda i: (i, 0))


def _full_spec(rows, width):
    return pl.BlockSpec((rows, width), lambda i: (0, 0))


def _tc1_body(d0, d1, x, w, hs_out, dinv_out):
    deg = d0[...] + d1[...] + 1.0
    dinv = lax.rsqrt(jnp.maximum(deg, 1.0))
    h = jnp.dot(x[...], w[...], preferred_element_type=jnp.float32)
    hs_out[...] = h * dinv[:, 0:1]
    dinv_out[...] = dinv


def _tc2_body(t0, t1, hs1, dinv, b1, w2, hs2_out):
    t = t0[...] + t1[...] + hs1[...]
    agg = dinv[:, 0:1] * t + b1[...]
    x1 = jnp.maximum(agg, 0.0)
    hs2_out[...] = jnp.dot(x1, w2[...], preferred_element_type=jnp.float32) * dinv[:, 0:1]


def _tc3_body(t0, t1, hs2, dinv, b2, x0, out):
    t = t0[...] + t1[...] + hs2[...]
    agg = dinv[:, 0:1] * t + b2[...]
    out[...] = (x0[...] + jnp.maximum(agg, 0.0)) * 0.5


def kernel(inputs, edge_index, W1, b1, W2, b2):
    e = edge_index.shape[1]
    n_chunks = -(-e // (NW * CH))
    e_pad = NW * n_chunks * CH
    pad = e_pad - e

    src = edge_index[0]
    dst = edge_index[1]
    if pad:
        src = jnp.concatenate([src, jnp.zeros((pad,), jnp.int32)])
        dst = jnp.concatenate([dst, N + (jnp.arange(pad, dtype=jnp.int32) % (N_PAD - N))])
    src3 = src.reshape(NW, n_chunks, CH)
    dst3 = dst.reshape(NW, n_chunks, CH)

    deg_k = _make_deg_kernel(n_chunks)
    edge_k = _make_edge_kernel(n_chunks)

    p = deg_k(dst3)
    p0 = p[0, :, :16]
    p1 = p[1, :, :16]

    b1r = b1.reshape(1, D)
    b2r = b2.reshape(1, D)

    hs1, dinv = pl.pallas_call(
        _tc1_body,
        grid=(N // BS,),
        in_specs=[_row_spec(16), _row_spec(16), _row_spec(D), _full_spec(D, D)],
        out_specs=[_row_spec(D), _row_spec(16)],
        out_shape=[
            jax.ShapeDtypeStruct((N, D), jnp.float32),
            jax.ShapeDtypeStruct((N, 16), jnp.float32),
        ],
    )(p0, p1, inputs, W1)

    t = edge_k(hs1, src3, dst3)
    t0, t1 = t[0], t[1]

    hs2 = pl.pallas_call(
        _tc2_body,
        grid=(N // BS,),
        in_specs=[_row_spec(D), _row_spec(D), _row_spec(D), _row_spec(16),
                  _full_spec(1, D), _full_spec(D, D)],
        out_specs=_row_spec(D),
        out_shape=jax.ShapeDtypeStruct((N, D), jnp.float32),
    )(t0, t1, hs1, dinv, b1r, W2)

    u = edge_k(hs2, src3, dst3)
    u0, u1 = u[0], u[1]

    out = pl.pallas_call(
        _tc3_body,
        grid=(N // BS,),
        in_specs=[_row_spec(D), _row_spec(D), _row_spec(D), _row_spec(16),
                  _full_spec(1, D), _row_spec(D)],
        out_specs=_row_spec(D),
        out_shape=jax.ShapeDtypeStruct((N, D), jnp.float32),
    )(u0, u1, hs2, dinv, b2r, inputs)

    return out
